# Initial kernel scaffold; baseline (speedup 1.0000x reference)
import jax
import jax.numpy as jnp
from jax.experimental import pallas as pl


def kernel(x, pos, edge_index, edge_labels, node_labels, W1, b1, W2, b2, Wr, br, Wfc1, Wfc2, Wfc3):
    raise NotImplementedError("write your pallas kernel here")


# speedup vs baseline: 6.0357x; 6.0357x over previous
import functools

import jax
import jax.numpy as jnp
from jax import lax
from jax.experimental import pallas as pl
from jax.experimental.pallas import tpu as pltpu
from jax.experimental.pallas import tpu_sc as plsc

NC = 2
NS = 16
BATCH = 128
BN = 1024


def _table_x_kernel(x_ref, w_ref, b_ref, out_ref):
    i = pl.program_id(0)
    nblk = pl.num_programs(0) - 1

    @pl.when(i < nblk)
    def _():
        h = x_ref[...]
        prod = h[:, None, :] * w_ref[...][None, :, :]
        out_ref[...] = prod.reshape(out_ref.shape)

    @pl.when(i == nblk)
    def _():
        nl = b_ref.shape[0]
        z = jnp.zeros((out_ref.shape[0] - nl, out_ref.shape[1]), jnp.float32)
        out_ref[...] = jnp.concatenate([b_ref[...], z], axis=0)


def _table_h_kernel(p0_ref, p1_ref, w_ref, b_ref, out_ref):
    i = pl.program_id(0)
    nblk = pl.num_programs(0) - 1

    @pl.when(i < nblk)
    def _():
        h = jnp.tanh(p0_ref[...] + p1_ref[...])
        prod = h[:, None, :] * w_ref[...][None, :, :]
        out_ref[...] = prod.reshape(out_ref.shape)

    @pl.when(i == nblk)
    def _():
        nl = b_ref.shape[0]
        z = jnp.zeros((out_ref.shape[0] - nl, out_ref.shape[1]), jnp.float32)
        out_ref[...] = jnp.concatenate([b_ref[...], z], axis=0)


def _build_table(h_or_p, W, b, *, from_partials):
    el, d = W.shape
    if from_partials:
        p0, p1 = h_or_p
        n = p0.shape[0]
    else:
        n = h_or_p.shape[0]
    nblk = n // BN
    rows = el * n + el * BN
    x_map = lambda i: (jnp.minimum(i, nblk - 1), 0)
    full2 = lambda i: (0, 0)
    if from_partials:
        body = _table_h_kernel
        in_specs = [
            pl.BlockSpec((BN, d), x_map),
            pl.BlockSpec((BN, d), x_map),
            pl.BlockSpec(W.shape, full2),
            pl.BlockSpec(b.shape, full2),
        ]
        args = (p0, p1, W, b)
    else:
        body = _table_x_kernel
        in_specs = [
            pl.BlockSpec((BN, d), x_map),
            pl.BlockSpec(W.shape, full2),
            pl.BlockSpec(b.shape, full2),
        ]
        args = (h_or_p, W, b)
    return pl.pallas_call(
        body,
        grid=(nblk + 1,),
        in_specs=in_specs,
        out_specs=pl.BlockSpec((el * BN, d), lambda i: (i, 0)),
        out_shape=jax.ShapeDtypeStruct((rows, d), jnp.float32),
    )(*args)


def _conv_sc(table, gidx, dstidx, n, d, nb):
    rows_per_tile = n // NS

    @functools.partial(
        pl.kernel,
        out_type=jax.ShapeDtypeStruct((NC, n, d), jnp.float32),
        mesh=plsc.VectorSubcoreMesh(core_axis_name="c", subcore_axis_name="s"),
        scratch_types=[
            pltpu.VMEM((nb, BATCH), jnp.int32),
            pltpu.VMEM((nb, BATCH), jnp.int32),
            pltpu.VMEM((BATCH, d), jnp.float32),
            pltpu.VMEM_SHARED((n, d), jnp.float32),
            pltpu.SemaphoreType.DMA,
        ],
    )
    def conv(table_hbm, gidx_hbm, dst_hbm, out_hbm,
             idx_v, dst_v, rbuf, acc, sem):
        c = lax.axis_index("c")
        s = lax.axis_index("s")
        w = c * NS + s

        def zrow(r, carry):
            for jj in range(d // 16):
                rbuf[r, pl.ds(jj * 16, 16)] = jnp.zeros((16,), jnp.float32)
            return carry
        lax.fori_loop(0, BATCH, zrow, 0)
        base = s * rows_per_tile
        for k in range(rows_per_tile // BATCH):
            pltpu.sync_copy(rbuf, acc.at[pl.ds(base + k * BATCH, BATCH)])
        plsc.subcore_barrier()

        pltpu.sync_copy(gidx_hbm.at[w], idx_v)
        pltpu.sync_copy(dst_hbm.at[w], dst_v)

        def step(j, carry):
            pltpu.async_copy(table_hbm.at[idx_v.at[j]], rbuf, sem).wait()
            pltpu.sync_copy(rbuf, acc.at[dst_v.at[j]], add=True)
            return carry
        lax.fori_loop(0, nb, step, 0)
        plsc.subcore_barrier()

        pltpu.sync_copy(acc.at[pl.ds(base, rows_per_tile)],
                        out_hbm.at[c, pl.ds(base, rows_per_tile)])

    return conv(table, gidx, dstidx)


def _head_kernel(p0_ref, p1_ref, oh_ref, wr_ref, br_ref,
                 w1_ref, w2_ref, w3_ref, out_ref):
    h = jnp.tanh(p0_ref[...] + p1_ref[...])
    nl = wr_ref.shape[0]
    hd = wr_ref.shape[2]
    acc = jnp.zeros((h.shape[0], hd), jnp.float32)
    for l in range(nl):
        y = lax.dot(h, wr_ref[l], preferred_element_type=jnp.float32)
        acc = acc + oh_ref[:, l:l + 1] * y
    z = jnp.tanh(acc + br_ref[...])
    z = jnp.tanh(lax.dot(z, w1_ref[...], preferred_element_type=jnp.float32))
    z = jnp.tanh(lax.dot(z, w2_ref[...], preferred_element_type=jnp.float32))
    out_ref[...] = jax.nn.sigmoid(
        lax.dot(z, w3_ref[...], preferred_element_type=jnp.float32))


def _head(p0, p1, oh, Wr, br, Wfc1, Wfc2, Wfc3):
    n, d = p0.shape
    nl, _, hd = Wr.shape
    nblk = n // BN
    blk2 = lambda i: (i, 0)
    full2 = lambda i: (0, 0)
    return pl.pallas_call(
        _head_kernel,
        grid=(nblk,),
        in_specs=[
            pl.BlockSpec((BN, d), blk2),
            pl.BlockSpec((BN, d), blk2),
            pl.BlockSpec((BN, nl), blk2),
            pl.BlockSpec(Wr.shape, lambda i: (0, 0, 0)),
            pl.BlockSpec((1, hd), full2),
            pl.BlockSpec(Wfc1.shape, full2),
            pl.BlockSpec(Wfc2.shape, full2),
            pl.BlockSpec(Wfc3.shape, full2),
        ],
        out_specs=pl.BlockSpec((BN, 1), blk2),
        out_shape=jax.ShapeDtypeStruct((n, 1), jnp.float32),
    )(p0, p1, oh, Wr, br.reshape(1, hd), Wfc1, Wfc2, Wfc3)


def kernel(x, pos, edge_index, edge_labels, node_labels,
           W1, b1, W2, b2, Wr, br, Wfc1, Wfc2, Wfc3):
    n, d = x.shape
    e = edge_index.shape[1]
    el = W1.shape[0]
    nl = b1.shape[0]
    hd = Wr.shape[2]

    src = edge_index[0].astype(jnp.int32)
    dst = edge_index[1].astype(jnp.int32)

    npad = -(-n // (NS * BATCH)) * (NS * BATCH)
    x = jnp.pad(x, ((0, npad - n), (0, 0)))

    ntiles = NC * NS
    total = e + n
    nb = -(-total // (ntiles * BATCH))
    cap = ntiles * nb * BATCH
    pad = cap - total
    gidx = jnp.concatenate([
        src * el + edge_labels.astype(jnp.int32),
        el * npad + node_labels.astype(jnp.int32),
        jnp.full((pad,), el * npad + nl, jnp.int32),
    ]).reshape(ntiles, nb, BATCH)
    dstidx = jnp.concatenate([
        dst,
        jnp.arange(n, dtype=jnp.int32),
        jnp.zeros((pad,), jnp.int32),
    ]).reshape(ntiles, nb, BATCH)

    table1 = _build_table(x, W1, b1, from_partials=False)
    p1 = _conv_sc(table1, gidx, dstidx, npad, d, nb)
    table2 = _build_table((p1[0], p1[1]), W2, b2, from_partials=True)
    p2 = _conv_sc(table2, gidx, dstidx, npad, d, nb)

    nlp = jnp.pad(node_labels, (0, npad - n), constant_values=nl)
    oh = (nlp[:, None] == jnp.arange(nl)[None, :]).astype(jnp.float32)
    return _head(p2[0], p2[1], oh, Wr, br, Wfc1, Wfc2, Wfc3)[:n]
